# baseline (speedup 1.0000x reference)
import functools

import numpy as np

import jax
import jax.numpy as jnp
from jax import lax
from jax.experimental import pallas as pl
from jax.experimental.pallas import tpu as pltpu
from jax.experimental.pallas import tpu_sc as plsc

N = 10000
D = 128
E = 320000
NC = 2
NS = 16
NW = NC * NS
CHUNK = 64
CHUNKS = 160
PHASES = 4
CPP = CHUNKS // PHASES
NBUF = 4
EPT = CHUNK * CHUNKS
E_PAD = EPT * NW
N_ACC = 10240
ROWS_PT = N_ACC // NS


def _sc_aggregate_body(with_deg, x_hbm, src_hbm, dst_hbm, z_hbm, z1_hbm,
                       agg_hbm, deg_hbm, src_v, dst_v, rows0_v, rows1_v,
                       rows2_v, rows3_v, ones_v, acc_sh, deg_sh,
                       gsem0, gsem1, gsem2, gsem3, ssem0, ssem1, ssem2, ssem3,
                       osem0, osem1, osem2, osem3):
    c = lax.axis_index("c")
    s = lax.axis_index("s")
    wid = c * NS + s
    r0 = s * ROWS_PT
    rows = (rows0_v, rows1_v, rows2_v, rows3_v)
    gsem = (gsem0, gsem1, gsem2, gsem3)
    ssem = (ssem0, ssem1, ssem2, ssem3)
    osem = (osem0, osem1, osem2, osem3)

    def gather(j, b):
        pltpu.async_copy(x_hbm.at[src_v.at[j]], rows[b], gsem[b])

    def wait_gather(j, b):
        pltpu.make_async_copy(x_hbm.at[src_v.at[j]], rows[b], gsem[b]).wait()

    def scatter(j, b):
        pltpu.async_copy(rows[b], acc_sh.at[dst_v.at[j]], ssem[b], add=True)
        if with_deg:
            pltpu.async_copy(ones_v, deg_sh.at[dst_v.at[j]], osem[b], add=True)

    def wait_scatter(j, b):
        pltpu.make_async_copy(rows[b], acc_sh.at[dst_v.at[j]], ssem[b]).wait()
        if with_deg:
            pltpu.make_async_copy(ones_v, deg_sh.at[dst_v.at[j]],
                                  osem[b]).wait()

    pltpu.sync_copy(src_hbm.at[wid, 0], src_v)
    pltpu.sync_copy(dst_hbm.at[wid, 0], dst_v)
    for b in range(NBUF):
        gather(b, b)

    if with_deg:
        for i in range(CHUNK // 16):
            ones_v[pl.ds(i * 16, 16)] = jnp.full((16,), 1.0, jnp.float32)
        if CHUNK % 16:
            ones_v[pl.ds(CHUNK - 16, 16)] = jnp.full((16,), 1.0, jnp.float32)

    pltpu.sync_copy(z_hbm.at[pl.ds(r0, ROWS_PT)], acc_sh.at[pl.ds(r0, ROWS_PT)])
    if with_deg:
        pltpu.sync_copy(z1_hbm.at[pl.ds(r0, ROWS_PT)],
                        deg_sh.at[pl.ds(r0, ROWS_PT)])

    plsc.subcore_barrier()

    for p in range(PHASES):
        if p > 0:
            pltpu.sync_copy(src_hbm.at[wid, p], src_v)
            pltpu.sync_copy(dst_hbm.at[wid, p], dst_v)
            for b in range(NBUF):
                gather(b, b)

        @pl.loop(0, CPP // NBUF - 1)
        def _quad(i):
            j0 = NBUF * i
            for k in range(NBUF):
                wait_gather(j0 + k, k)
                scatter(j0 + k, k)
            for k in range(NBUF):
                wait_scatter(j0 + k, k)
                gather(j0 + k + NBUF, k)

        j0 = CPP - NBUF
        for k in range(NBUF):
            wait_gather(j0 + k, k)
            scatter(j0 + k, k)
        for k in range(NBUF):
            wait_scatter(j0 + k, k)

    plsc.subcore_barrier()

    pltpu.sync_copy(acc_sh.at[pl.ds(r0, ROWS_PT)],
                    agg_hbm.at[c, pl.ds(r0, ROWS_PT)])
    if with_deg:
        pltpu.sync_copy(deg_sh.at[pl.ds(r0, ROWS_PT)],
                        deg_hbm.at[c, pl.ds(r0, ROWS_PT)])


@functools.cache
def _make_sc_aggregate(with_deg):
  return pl.kernel(
    functools.partial(_sc_aggregate_body, with_deg),
    out_type=(
        jax.ShapeDtypeStruct((NC, N_ACC, D), jnp.float32),
        jax.ShapeDtypeStruct((NC, N_ACC), jnp.float32),
    ),
    mesh=plsc.VectorSubcoreMesh(core_axis_name="c", subcore_axis_name="s",
                                num_cores=NC, num_subcores=NS),
    scratch_types=[
        pltpu.VMEM((CPP, CHUNK), jnp.int32),
        pltpu.VMEM((CPP, CHUNK), jnp.int32),
        pltpu.VMEM((CHUNK, D), jnp.float32),
        pltpu.VMEM((CHUNK, D), jnp.float32),
        pltpu.VMEM((CHUNK, D), jnp.float32),
        pltpu.VMEM((CHUNK, D), jnp.float32),
        pltpu.VMEM((CHUNK,), jnp.float32),
        pltpu.VMEM_SHARED((N_ACC, D), jnp.float32),
        pltpu.VMEM_SHARED((N_ACC,), jnp.float32),
    ] + [pltpu.SemaphoreType.DMA] * 12,
    name="sc_sage_agg_deg" if with_deg else "sc_sage_agg",
  )

BM = 2000


def _tc_layer_body(x_ref, agg_ref, rdeg_ref, ws_ref, wa_ref, o_ref):
    agg = (agg_ref[0] + agg_ref[1]) * rdeg_ref[...]
    h = jnp.dot(x_ref[...], ws_ref[...], preferred_element_type=jnp.float32)
    h = h + jnp.dot(agg, wa_ref[...], preferred_element_type=jnp.float32)
    h = jnp.maximum(h, 0.0)
    ss = jnp.maximum(jnp.sum(h * h, axis=1, keepdims=True), 1e-24)
    o_ref[...] = h * lax.rsqrt(ss)


def _tc_layer(x, agg_parts, rdeg, w):
    ws, wa = w[:D], w[D:]
    return pl.pallas_call(
        _tc_layer_body,
        grid=(N // BM,),
        in_specs=[
            pl.BlockSpec((BM, D), lambda i: (i, 0)),
            pl.BlockSpec((NC, BM, D), lambda i: (0, i, 0)),
            pl.BlockSpec((BM, 1), lambda i: (i, 0)),
            pl.BlockSpec((D, D), lambda i: (0, 0)),
            pl.BlockSpec((D, D), lambda i: (0, 0)),
        ],
        out_specs=pl.BlockSpec((BM, D), lambda i: (i, 0)),
        out_shape=jax.ShapeDtypeStruct((N, D), jnp.float32),
    )(x, agg_parts, rdeg, ws, wa)


_ZEROS2D = np.zeros((N_ACC, D), np.float32)
_ZEROS1D = np.zeros((N_ACC,), np.float32)


@jax.jit
def kernel(x, adj, past, W1, W2):
    del past
    zeros2d = _ZEROS2D
    zeros1d = _ZEROS1D

    pad_src = jnp.arange(E_PAD - E, dtype=jnp.int32) * 997 % N
    src = jnp.concatenate([adj[0], pad_src]).reshape(NW, PHASES, CPP, CHUNK)
    pad_dst = N + jnp.arange(E_PAD - E, dtype=jnp.int32) % (N_ACC - N)
    dst = jnp.concatenate([adj[1], pad_dst]).reshape(NW, PHASES, CPP, CHUNK)

    agg1, deg = _make_sc_aggregate(True)(x, src, dst, zeros2d, zeros1d)
    rdeg = (1.0 / jnp.maximum(deg[0, :N] + deg[1, :N], 1.0)).reshape(N, 1)
    h = _tc_layer(x, agg1, rdeg, W1)
    agg2, _ = _make_sc_aggregate(False)(h, src, dst, zeros2d, zeros1d)
    feat = _tc_layer(h, agg2, rdeg, W2)
    return feat
